# baseline (speedup 1.0000x reference)
import jax
import jax.numpy as jnp
from jax.experimental import pallas as pl
from jax.experimental.pallas import tpu as pltpu

_N = 4096
_R = 256
_KNN = 8
_INF = float("inf")


def _dist_block(rows, alln, diag, d_ref):
    g = jax.lax.dot_general(rows, alln, (((1,), (1,)), ((), ())),
                            preferred_element_type=jnp.float32)
    sq_r = jnp.sum(rows * rows, axis=1, keepdims=True)
    ones = jnp.ones((1, rows.shape[1]), jnp.float32)
    sq_a = jax.lax.dot_general(ones, alln * alln, (((1,), (1,)), ((), ())),
                               preferred_element_type=jnp.float32)
    d_ref[...] = jnp.where(diag, _INF, (sq_r + sq_a) - 2.0 * g)


def _mask_topk(d_ref, cols):
    idx = jnp.argmin(d_ref[...], axis=1, keepdims=True)
    for _ in range(_KNN - 1):
        d = jnp.where(cols == idx, _INF, d_ref[...])
        d_ref[...] = d
        idx = jnp.argmin(d, axis=1, keepdims=True)
    return idx


def _body(xr, xa, zr, za, out_ref, dx_ref, dz_ref):
    i = pl.program_id(0)
    cols = jax.lax.broadcasted_iota(jnp.int32, (_R, _N), 1)
    row_g = i * _R + jax.lax.broadcasted_iota(jnp.int32, (_R, _N), 0)
    diag = cols == row_g

    _dist_block(xr[...], xa[...], diag, dx_ref)
    ix_last = _mask_topk(dx_ref, cols)

    _dist_block(zr[...], za[...], diag, dz_ref)
    iz_last = _mask_topk(dz_ref, cols)

    sel_x = (dx_ref[...] == _INF) | (cols == ix_last)
    sel_z = (dz_ref[...] == _INF) | (cols == iz_last)
    both = sel_x & sel_z & jnp.logical_not(diag)
    c = jnp.sum(both.astype(jnp.float32))
    out_ref[...] = jnp.full((8, 128), (16.0 * _R - 2.0 * c) * (100.0 / (_N * _N)),
                            jnp.float32)


def kernel(X, Z):
    n, dx = X.shape
    _, dz = Z.shape
    out = pl.pallas_call(
        _body,
        grid=(n // _R,),
        in_specs=[
            pl.BlockSpec((_R, dx), lambda i: (i, 0)),
            pl.BlockSpec((n, dx), lambda i: (0, 0)),
            pl.BlockSpec((_R, dz), lambda i: (i, 0)),
            pl.BlockSpec((n, dz), lambda i: (0, 0)),
        ],
        out_specs=pl.BlockSpec((8, 128), lambda i: (i, 0)),
        out_shape=jax.ShapeDtypeStruct((8 * (n // _R), 128), jnp.float32),
        scratch_shapes=[
            pltpu.VMEM((_R, _N), jnp.float32),
            pltpu.VMEM((_R, _N), jnp.float32),
        ],
        compiler_params=pltpu.CompilerParams(
            dimension_semantics=("parallel",)),
    )(X, X, Z, Z)
    return jnp.sum(out[::8, 0])


# speedup vs baseline: 2.5711x; 2.5711x over previous
import jax
import jax.numpy as jnp
from jax.experimental import pallas as pl
from jax.experimental.pallas import tpu as pltpu

_N = 4096
_R = 256
_LANES = 128
_NS = _N // _LANES
_INF = float("inf")

_BATCHER8 = [(0, 1), (2, 3), (4, 5), (6, 7),
             (0, 2), (1, 3), (1, 2),
             (4, 6), (5, 7), (5, 6),
             (0, 4), (1, 5), (2, 6), (3, 7),
             (2, 4), (3, 5),
             (1, 2), (3, 4), (5, 6)]
_BITONIC8 = [(0, 4), (1, 5), (2, 6), (3, 7),
             (0, 2), (1, 3), (4, 6), (5, 7),
             (0, 1), (2, 3), (4, 5), (6, 7)]


def _dist_block(rows, alln, diag, d_ref):
    g = jax.lax.dot_general(rows, alln, (((1,), (1,)), ((), ())),
                            preferred_element_type=jnp.float32)
    sq_r = jnp.sum(rows * rows, axis=1, keepdims=True)
    ones = jnp.ones((1, rows.shape[1]), jnp.float32)
    sq_a = jax.lax.dot_general(ones, alln * alln, (((1,), (1,)), ((), ())),
                               preferred_element_type=jnp.float32)
    d_ref[...] = jnp.where(diag, _INF, (sq_r + sq_a) - 2.0 * g)


def _ce(a, i, j):
    lo = jnp.minimum(a[i], a[j])
    hi = jnp.maximum(a[i], a[j])
    a[i], a[j] = lo, hi


def _merge8(a, b, sort=True):
    m = [jnp.minimum(a[i], b[7 - i]) for i in range(8)]
    if sort:
        for i, j in _BITONIC8:
            _ce(m, i, j)
    return m


def _top8_threshold(d_ref):
    v = [d_ref[:, _LANES * k:_LANES * (k + 1)] for k in range(_NS)]
    groups = []
    for g in range(4):
        a = list(v[8 * g:8 * g + 8])
        for i, j in _BATCHER8:
            _ce(a, i, j)
        groups.append(a)
    m1 = _merge8(groups[0], groups[1])
    m2 = _merge8(groups[2], groups[3])
    f = _merge8(m1, m2, sort=False)
    t = None
    for r in range(8):
        m8 = f[0]
        for i in range(1, 8):
            m8 = jnp.minimum(m8, f[i])
        t = jnp.min(m8, axis=1, keepdims=True)
        if r < 7:
            f = [jnp.where(fi == t, _INF, fi) for fi in f]
    return t


def _body(xr, xa, zr, za, out_ref, dx_ref, dz_ref):
    i = pl.program_id(0)
    cols = jax.lax.broadcasted_iota(jnp.int32, (_R, _N), 1)
    row_g = i * _R + jax.lax.broadcasted_iota(jnp.int32, (_R, _N), 0)
    diag = cols == row_g

    _dist_block(xr[...], xa[...], diag, dx_ref)
    _dist_block(zr[...], za[...], diag, dz_ref)

    t8x = _top8_threshold(dx_ref)
    t8z = _top8_threshold(dz_ref)

    both = (dx_ref[...] <= t8x) & (dz_ref[...] <= t8z)
    c = jnp.sum(both.astype(jnp.float32))
    out_ref[...] = jnp.full((8, 128), (16.0 * _R - 2.0 * c) * (100.0 / (_N * _N)),
                            jnp.float32)


def kernel(X, Z):
    n, dx = X.shape
    _, dz = Z.shape
    out = pl.pallas_call(
        _body,
        grid=(n // _R,),
        in_specs=[
            pl.BlockSpec((_R, dx), lambda i: (i, 0)),
            pl.BlockSpec((n, dx), lambda i: (0, 0)),
            pl.BlockSpec((_R, dz), lambda i: (i, 0)),
            pl.BlockSpec((n, dz), lambda i: (0, 0)),
        ],
        out_specs=pl.BlockSpec((8, 128), lambda i: (i, 0)),
        out_shape=jax.ShapeDtypeStruct((8 * (n // _R), 128), jnp.float32),
        scratch_shapes=[
            pltpu.VMEM((_R, _N), jnp.float32),
            pltpu.VMEM((_R, _N), jnp.float32),
        ],
        compiler_params=pltpu.CompilerParams(
            dimension_semantics=("parallel",)),
    )(X, X, Z, Z)
    return jnp.sum(out[::8, 0])
